# Initial kernel scaffold; baseline (speedup 1.0000x reference)
import jax
import jax.numpy as jnp
from jax.experimental import pallas as pl


def kernel(x_paper, x_author, edge_index_a2p, edge_index_p2a, lin_paper_W, lin_paper_b, lin_author_W, lin_author_b, Wl0_a2p, bl0_a2p, Wr0_a2p, Wl0_p2a, bl0_p2a, Wr0_p2a, Wl1_a2p, bl1_a2p, Wr1_a2p, Wl1_p2a, bl1_p2a, Wr1_p2a):
    raise NotImplementedError("write your pallas kernel here")


# speedup vs baseline: 6.8437x; 6.8437x over previous
import functools

import jax
import jax.numpy as jnp
from jax import lax
from jax.experimental import pallas as pl
from jax.experimental.pallas import tpu as pltpu
from jax.experimental.pallas import tpu_sc as plsc

N_NODES = 10000
FDIM = 128
NC, NS = 2, 16
CH = 128
ACC_ROWS = 10240
ZROWS = ACC_ROWS // NS
OUT_PER_W = 1000
CNT_PER_W = 1000


_BLK = 1000
_GRID = N_NODES // _BLK


def _row_spec():
    return pl.BlockSpec((_BLK, FDIM), lambda i: (i, 0))


def _w_spec():
    return pl.BlockSpec((FDIM, FDIM), lambda i: (0, 0))


def _b_spec():
    return pl.BlockSpec((1, FDIM), lambda i: (0, 0))


def _cnt_spec():
    return pl.BlockSpec((_BLK, 1), lambda i: (i, 0))


def _mm(x, w):
    return jnp.dot(x, w, preferred_element_type=jnp.float32)


def _stage_a_body(xp_ref, xa_ref, wp_ref, bp_ref, wa_ref, ba_ref,
                  wl_a2p_ref, wl_p2a_ref, wr_a2p_ref, bl_a2p_ref,
                  wr_p2a_ref, bl_p2a_ref,
                  ta_ref, tp_ref, rp_ref, ra_ref):
    xp = jnp.maximum(_mm(xp_ref[...], wp_ref[...]) + bp_ref[...], 0.0)
    xa = jnp.maximum(_mm(xa_ref[...], wa_ref[...]) + ba_ref[...], 0.0)
    ta_ref[...] = _mm(xa, wl_a2p_ref[...])
    tp_ref[...] = _mm(xp, wl_p2a_ref[...])
    rp_ref[...] = _mm(xp, wr_a2p_ref[...]) + bl_a2p_ref[...]
    ra_ref[...] = _mm(xa, wr_p2a_ref[...]) + bl_p2a_ref[...]


def _stage_a(x_paper, x_author, wp, bp, wa, ba, wl_a2p, wl_p2a,
             wr_a2p, bl_a2p, wr_p2a, bl_p2a):
    out = jax.ShapeDtypeStruct((N_NODES, FDIM), jnp.float32)
    return pl.pallas_call(
        _stage_a_body,
        grid=(_GRID,),
        in_specs=[_row_spec(), _row_spec(), _w_spec(), _b_spec(), _w_spec(),
                  _b_spec(), _w_spec(), _w_spec(), _w_spec(), _b_spec(),
                  _w_spec(), _b_spec()],
        out_specs=[_row_spec(), _row_spec(), _row_spec(), _row_spec()],
        out_shape=[out, out, out, out],
    )(x_paper, x_author, wp, bp, wa, ba, wl_a2p, wl_p2a, wr_a2p, bl_a2p,
      wr_p2a, bl_p2a)


def _stage_c_body(sp_ref, sa_ref, cp_ref, ca_ref, rp_ref, ra_ref,
                  wl_a2p_ref, wl_p2a_ref, wr_a2p_ref, bl_a2p_ref,
                  wr_p2a_ref, bl_p2a_ref,
                  ta_ref, tp_ref, rp1_ref, ra1_ref):
    xp1 = sp_ref[...] / jnp.maximum(cp_ref[...], 1.0) + rp_ref[...]
    xa1 = sa_ref[...] / jnp.maximum(ca_ref[...], 1.0) + ra_ref[...]
    ta_ref[...] = _mm(xa1, wl_a2p_ref[...])
    tp_ref[...] = _mm(xp1, wl_p2a_ref[...])
    rp1_ref[...] = _mm(xp1, wr_a2p_ref[...]) + bl_a2p_ref[...]
    ra1_ref[...] = _mm(xa1, wr_p2a_ref[...]) + bl_p2a_ref[...]


def _stage_c(sp, sa, cp, ca, rp, ra, wl_a2p, wl_p2a, wr_a2p, bl_a2p,
             wr_p2a, bl_p2a):
    out = jax.ShapeDtypeStruct((N_NODES, FDIM), jnp.float32)
    return pl.pallas_call(
        _stage_c_body,
        grid=(_GRID,),
        in_specs=[_row_spec(), _row_spec(), _cnt_spec(), _cnt_spec(),
                  _row_spec(), _row_spec(), _w_spec(), _w_spec(), _w_spec(),
                  _b_spec(), _w_spec(), _b_spec()],
        out_specs=[_row_spec(), _row_spec(), _row_spec(), _row_spec()],
        out_shape=[out, out, out, out],
    )(sp, sa, cp, ca, rp, ra, wl_a2p, wl_p2a, wr_a2p, bl_a2p, wr_p2a, bl_p2a)


def _stage_d_body(sp_ref, sa_ref, cp_ref, ca_ref, rp_ref, ra_ref,
                  xp2_ref, xa2_ref):
    xp2_ref[...] = sp_ref[...] / jnp.maximum(cp_ref[...], 1.0) + rp_ref[...]
    xa2_ref[...] = sa_ref[...] / jnp.maximum(ca_ref[...], 1.0) + ra_ref[...]


def _stage_d(sp, sa, cp, ca, rp, ra):
    out = jax.ShapeDtypeStruct((N_NODES, FDIM), jnp.float32)
    return pl.pallas_call(
        _stage_d_body,
        grid=(_GRID,),
        in_specs=[_row_spec(), _row_spec(), _cnt_spec(), _cnt_spec(),
                  _row_spec(), _row_spec()],
        out_specs=[_row_spec(), _row_spec()],
        out_shape=[out, out],
    )(sp, sa, cp, ca, rp, ra)


def _one_core(table, src3, dst3, out_hbm, cnt_out, zrows, zcnt, ones_hbm,
              acc, cnt_sh, src_v, dst_v, rows, ones_v, cnt_v, w, k,
              with_counts):
    pltpu.sync_copy(zrows, acc.at[pl.ds(w * ZROWS, ZROWS)])
    pltpu.sync_copy(src3.at[w], src_v)
    pltpu.sync_copy(dst3.at[w], dst_v)
    if with_counts:
        pltpu.sync_copy(zcnt, cnt_sh.at[pl.ds(w * ZROWS, ZROWS)])
        pltpu.sync_copy(ones_hbm, ones_v)
    plsc.subcore_barrier()

    @pl.loop(0, k)
    def _(j):
        pltpu.sync_copy(table.at[src_v.at[j]], rows)
        pltpu.sync_copy(rows, acc.at[dst_v.at[j]], add=True)
        if with_counts:
            pltpu.sync_copy(ones_v, cnt_sh.at[dst_v.at[j]], add=True)

    plsc.subcore_barrier()

    @pl.when(w < N_NODES // OUT_PER_W)
    def _():
        pltpu.sync_copy(acc.at[pl.ds(w * OUT_PER_W, OUT_PER_W)],
                        out_hbm.at[pl.ds(w * OUT_PER_W, OUT_PER_W)])
    if with_counts:
        @pl.when(w < N_NODES // CNT_PER_W)
        def _():
            pltpu.sync_copy(cnt_sh.at[pl.ds(w * CNT_PER_W, CNT_PER_W)], cnt_v)
            pltpu.sync_copy(cnt_v, cnt_out.at[pl.ds(w * CNT_PER_W, CNT_PER_W)])


def _make_sc_agg(k, with_counts):
    mesh = plsc.VectorSubcoreMesh(core_axis_name="c", subcore_axis_name="s",
                                  num_cores=NC, num_subcores=NS)
    frow = jax.ShapeDtypeStruct((N_NODES, FDIM), jnp.float32)
    fcnt = jax.ShapeDtypeStruct((N_NODES,), jnp.float32)
    out_type = (frow, frow, fcnt, fcnt) if with_counts else (frow, frow)

    @functools.partial(
        pl.kernel, mesh=mesh, out_type=out_type,
        scratch_types=[
            pltpu.VMEM_SHARED((ACC_ROWS, FDIM), jnp.float32),
            pltpu.VMEM_SHARED((ACC_ROWS,), jnp.float32),
            pltpu.VMEM((k, CH), jnp.int32),
            pltpu.VMEM((k, CH), jnp.int32),
            pltpu.VMEM((CH, FDIM), jnp.float32),
            pltpu.VMEM((CH,), jnp.float32),
            pltpu.VMEM((CNT_PER_W,), jnp.float32),
        ],
    )
    def agg(ta, tp, src_a2p, dst_a2p, src_p2a, dst_p2a, zrows, zcnt, ones_hbm,
            *rest):
        if with_counts:
            sp_out, sa_out, cp_out, ca_out = rest[:4]
            scratch = rest[4:]
        else:
            sp_out, sa_out = rest[:2]
            cp_out = ca_out = None
            scratch = rest[2:]
        acc, cnt_sh, src_v, dst_v, rows, ones_v, cnt_v = scratch
        cid = lax.axis_index("c")
        w = lax.axis_index("s")

        @pl.when(cid == 0)
        def _():
            _one_core(ta, src_a2p, dst_a2p, sp_out, cp_out, zrows, zcnt,
                      ones_hbm, acc, cnt_sh, src_v, dst_v, rows, ones_v,
                      cnt_v, w, k, with_counts)

        @pl.when(cid == 1)
        def _():
            _one_core(tp, src_p2a, dst_p2a, sa_out, ca_out, zrows, zcnt,
                      ones_hbm, acc, cnt_sh, src_v, dst_v, rows, ones_v,
                      cnt_v, w, k, with_counts)

    return agg


def _prep_edges(ei):
    e = ei.shape[1]
    k = -(-e // (NS * CH))
    epad = NS * CH * k
    pad = epad - e
    src = ei[0].astype(jnp.int32)
    dst = ei[1].astype(jnp.int32)
    pidx = jnp.arange(pad, dtype=jnp.int32)
    src = jnp.concatenate([src, pidx % N_NODES])
    dst = jnp.concatenate([dst, N_NODES + pidx % (ACC_ROWS - N_NODES)])
    return src.reshape(NS, k, CH), dst.reshape(NS, k, CH), k


def kernel(x_paper, x_author, edge_index_a2p, edge_index_p2a,
           lin_paper_W, lin_paper_b, lin_author_W, lin_author_b,
           Wl0_a2p, bl0_a2p, Wr0_a2p, Wl0_p2a, bl0_p2a, Wr0_p2a,
           Wl1_a2p, bl1_a2p, Wr1_a2p, Wl1_p2a, bl1_p2a, Wr1_p2a):
    b2 = lambda b: b.reshape(1, FDIM)
    src_a2p, dst_a2p, k = _prep_edges(edge_index_a2p)
    src_p2a, dst_p2a, _ = _prep_edges(edge_index_p2a)
    zrows = jnp.zeros((ZROWS, FDIM), jnp.float32)
    zcnt = jnp.zeros((ZROWS,), jnp.float32)
    ones = jnp.ones((CH,), jnp.float32)

    agg0 = _make_sc_agg(k, with_counts=True)
    agg1 = _make_sc_agg(k, with_counts=False)

    ta0, tp0, rp0, ra0 = _stage_a(
        x_paper, x_author, lin_paper_W, b2(lin_paper_b), lin_author_W,
        b2(lin_author_b), Wl0_a2p, Wl0_p2a, Wr0_a2p, b2(bl0_a2p), Wr0_p2a,
        b2(bl0_p2a))

    sp0, sa0, cnt_p, cnt_a = agg0(ta0, tp0, src_a2p, dst_a2p, src_p2a,
                                  dst_p2a, zrows, zcnt, ones)
    cp = cnt_p.reshape(N_NODES, 1)
    ca = cnt_a.reshape(N_NODES, 1)

    ta1, tp1, rp1, ra1 = _stage_c(
        sp0, sa0, cp, ca, rp0, ra0, Wl1_a2p, Wl1_p2a, Wr1_a2p, b2(bl1_a2p),
        Wr1_p2a, b2(bl1_p2a))

    sp1, sa1 = agg1(ta1, tp1, src_a2p, dst_a2p, src_p2a, dst_p2a, zrows,
                    zcnt, ones)

    xp2, xa2 = _stage_d(sp1, sa1, cp, ca, rp1, ra1)
    return (xp2, xa2)
